# baseline (speedup 1.0000x reference)
import functools

import jax
import jax.numpy as jnp
from jax import lax
from jax.experimental import pallas as pl
from jax.experimental.pallas import tpu as pltpu
from jax.experimental.pallas import tpu_sc as plsc

_S = 30.0
_M = 0.5
_STD = 0.0125
_SHIFT = 30.0
_LOG2E = 1.4426950408889634
_A = _S * _LOG2E
_BB = _SHIFT * _LOG2E

_K_SC = 36864
_NC, _NS = 2, 16
_NW = _NC * _NS
_CR = 32
_VEC = 16
_CK = 2048


def _tc_kernel(x_ref, out_ref, sum_ref, *, n_classes, cb_off):
    cb = pl.program_id(0)
    ncb = pl.num_programs(0)

    @pl.when(cb == 0)
    def _init():
        sum_ref[...] = jnp.zeros_like(sum_ref)

    x = x_ref[...]
    k, b = x.shape
    e = jnp.exp2(x * _A - _BB)

    @pl.when(cb != ncb - 1)
    def _body():
        sum_ref[0:1, :] += jnp.sum(e, axis=0, keepdims=True)

    @pl.when(cb == ncb - 1)
    def _last():
        cls = jax.lax.broadcasted_iota(jnp.int32, (k, b), 0) \
            + (cb + cb_off) * _CK
        sum_ref[0:1, :] += jnp.sum(jnp.where(cls < n_classes, e, 0.0),
                                   axis=0, keepdims=True)
        out_ref[...] = sum_ref[0:1, :]


def _epilogue_kernel(tc_ref, sc_ref, xlab_ref, margin_ref, out_ref):
    xl = xlab_ref[...]
    e_lab = jnp.exp2(xl * _A - _BB)
    xlc = jnp.clip(xl, -1.0 + 1e-7, 1.0 - 1e-7)
    sin_theta = jnp.sqrt(jnp.maximum(1.0 - xlc * xlc, 0.0))
    mg = margin_ref[...]
    mprime = (xlc * jnp.cos(mg) - sin_theta * jnp.sin(mg)) * _S
    total = tc_ref[...] + jnp.sum(sc_ref[...], axis=0, keepdims=True) \
        - e_lab + jnp.exp2(mprime * _LOG2E - _BB)
    loss = jnp.log(total) + _SHIFT - mprime
    out_ref[...] = jnp.sum(loss, axis=1, keepdims=True) / loss.shape[1]


def _make_sc_part(n_batch):
    span = _K_SC // _NW
    n_chunks = span // _CR
    bpw = n_batch // _NW
    nbv = n_batch // _VEC
    mesh = plsc.VectorSubcoreMesh(core_axis_name="c", subcore_axis_name="s")

    @functools.partial(
        pl.kernel,
        mesh=mesh,
        out_type=(
            jax.ShapeDtypeStruct((_NW * n_batch,), jnp.float32),
            jax.ShapeDtypeStruct((n_batch,), jnp.float32),
        ),
        scratch_types=[
            pltpu.VMEM((2, _CR, n_batch), jnp.float32),
            pltpu.VMEM((n_batch,), jnp.float32),
            pltpu.VMEM((bpw,), jnp.float32),
            pltpu.VMEM((bpw,), jnp.float32),
            pltpu.VMEM((bpw, 8, 128), jnp.float32),
            pltpu.VMEM((bpw,), jnp.float32),
            pltpu.SemaphoreType.DMA((2,)),
            pltpu.SemaphoreType.DMA,
        ],
        compiler_params=pltpu.CompilerParams(needs_layout_passes=False),
    )
    def _sc(x_hbm, g8_hbm, sub8_hbm, sums_hbm, xlab_hbm, buf, acc, g8v,
            sub8v, ltile, xstage, sems, lsem):
        wid = lax.axis_index("s") * _NC + lax.axis_index("c")
        r0 = wid * span
        b32 = wid * bpw
        b0 = (b32 // 128) * 128
        iota = lax.iota(jnp.int32, _VEC)

        pltpu.sync_copy(g8_hbm.at[pl.ds(b32, bpw)], g8v)
        pltpu.sync_copy(sub8_hbm.at[pl.ds(b32, bpw)], sub8v)

        def _scalar_at(ref, r):
            half = ref[pl.ds((r // _VEC) * _VEC, _VEC)]
            return jnp.sum(jnp.where(iota == lax.rem(r, _VEC), half, 0.0))

        @pl.loop(0, bpw)
        def _fire(r):
            g8 = pl.multiple_of(_scalar_at(g8v, r).astype(jnp.int32), 8)
            pltpu.async_copy(x_hbm.at[pl.ds(g8, 8), pl.ds(b0, 128)],
                             ltile.at[r], lsem)

        @pl.loop(0, nbv)
        def _zero(j):
            acc[pl.ds(j * _VEC, _VEC)] = jnp.zeros((_VEC,), jnp.float32)

        pltpu.async_copy(
            x_hbm.at[pl.ds(r0, _CR), :], buf.at[0], sems.at[0])

        @pl.loop(0, n_chunks)
        def _chunks(t):
            slot = lax.rem(t, 2)

            @pl.when(t + 1 < n_chunks)
            def _prefetch():
                nslot = lax.rem(t + 1, 2)
                row = pl.multiple_of(r0 + (t + 1) * _CR, 8)
                pltpu.async_copy(x_hbm.at[pl.ds(row, _CR), :],
                                 buf.at[nslot], sems.at[nslot])

            pltpu.make_async_copy(
                x_hbm.at[pl.ds(r0, _CR), :],
                buf.at[slot], sems.at[slot]).wait()

            @pl.loop(0, nbv)
            def _bv(j):
                zero8 = (jnp.zeros((_VEC,), jnp.float32),) * 8

                def _acc_body(r, c):
                    col = pl.ds(j * _VEC, _VEC)
                    return tuple(
                        a + jnp.exp(buf[slot, r + i, col] * _S - _SHIFT)
                        for i, a in enumerate(c))

                a = plsc.parallel_loop(
                    0, _CR, 8, unroll=1, carry=zero8)(_acc_body)
                col = pl.ds(j * _VEC, _VEC)
                acc[col] = acc[col] + (((a[0] + a[1]) + (a[2] + a[3]))
                                       + ((a[4] + a[5]) + (a[6] + a[7])))

        pltpu.sync_copy(acc, sums_hbm.at[pl.ds(wid * n_batch, n_batch)])

        @pl.loop(0, bpw)
        def _drain(r):
            pltpu.make_async_copy(x_hbm.at[pl.ds(0, 8), pl.ds(0, 128)],
                                  ltile.at[r], lsem).wait()

        zerox = (jnp.zeros((_VEC,), jnp.float32),
                 jnp.zeros((_VEC,), jnp.float32))

        @pl.loop(0, bpw, init_carry=zerox)
        def xvecs(r, xv):
            x0, x1 = xv
            sub8 = _scalar_at(sub8v, r).astype(jnp.int32)
            colv = ((b32 + r - b0) // _VEC) * _VEC
            v = ltile[r, sub8, pl.ds(colv, _VEC)]
            lane_hit = iota == lax.rem(r, _VEC)
            xl_r = jnp.sum(jnp.where(lane_hit, v, 0.0))
            in0 = r // _VEC == 0
            x0 = jnp.where(in0 & lane_hit, xl_r, x0)
            x1 = jnp.where(jnp.logical_not(in0) & lane_hit, xl_r, x1)
            return (x0, x1)

        xstage[pl.ds(0, _VEC)] = xvecs[0]
        xstage[pl.ds(_VEC, _VEC)] = xvecs[1]
        pltpu.sync_copy(xstage, xlab_hbm.at[pl.ds(b32, bpw)])

    return _sc


@jax.jit
def kernel(input, label):
    b, c = input.shape
    xt = input.T
    cb_off = _K_SC // _CK
    n_cb = pl.cdiv(c - _K_SC, _CK)

    margin = _M + _STD * jax.random.normal(jax.random.key(42), (b,),
                                           dtype=jnp.float32)
    valid = label != -1
    margin = jnp.where(valid, margin, 0.0)
    safe_label = jnp.where(valid, label, 0).astype(jnp.int32)

    g8_f = ((safe_label // 8) * 8).astype(jnp.float32)
    sub8_f = (safe_label % 8).astype(jnp.float32)

    sc_parts, sc_xlab = _make_sc_part(b)(xt, g8_f, sub8_f)

    tc_sums = pl.pallas_call(
        functools.partial(_tc_kernel, n_classes=c, cb_off=cb_off),
        grid=(n_cb,),
        in_specs=[pl.BlockSpec((_CK, b), lambda cb: (cb + cb_off, 0))],
        out_specs=pl.BlockSpec((1, b), lambda cb: (0, 0)),
        out_shape=jax.ShapeDtypeStruct((1, b), jnp.float32),
        scratch_shapes=[pltpu.VMEM((8, b), jnp.float32)],
        compiler_params=pltpu.CompilerParams(
            dimension_semantics=("arbitrary",),
        ),
    )(xt)

    loss_mean = pl.pallas_call(
        _epilogue_kernel,
        in_specs=[
            pl.BlockSpec((1, b), lambda: (0, 0)),
            pl.BlockSpec((_NW, b), lambda: (0, 0)),
            pl.BlockSpec((1, b), lambda: (0, 0)),
            pl.BlockSpec((1, b), lambda: (0, 0)),
        ],
        out_specs=pl.BlockSpec((1, 1), lambda: (0, 0)),
        out_shape=jax.ShapeDtypeStruct((1, 1), jnp.float32),
    )(tc_sums, sc_parts.reshape(_NW, b), sc_xlab[None, :], margin[None, :])

    return loss_mean[0, 0]


# speedup vs baseline: 1.0230x; 1.0230x over previous
import functools

import jax
import jax.numpy as jnp
from jax import lax
from jax.experimental import pallas as pl
from jax.experimental.pallas import tpu as pltpu
from jax.experimental.pallas import tpu_sc as plsc

_S = 30.0
_M = 0.5
_STD = 0.0125
_SHIFT = 30.0
_LOG2E = 1.4426950408889634
_A = _S * _LOG2E
_BB = _SHIFT * _LOG2E

_K_SC = 36864
_NC, _NS = 2, 16
_NW = _NC * _NS
_CR = 32
_VEC = 16
_CK = 3072


def _tc_kernel(x_ref, out_ref, sum_ref, *, n_classes, cb_off):
    cb = pl.program_id(0)
    ncb = pl.num_programs(0)

    @pl.when(cb == 0)
    def _init():
        sum_ref[...] = jnp.zeros_like(sum_ref)

    x = x_ref[...]
    k, b = x.shape
    e = jnp.exp2(x * _A - _BB)

    @pl.when(cb != ncb - 1)
    def _body():
        sum_ref[0:1, :] += jnp.sum(e, axis=0, keepdims=True)

    @pl.when(cb == ncb - 1)
    def _last():
        cls = jax.lax.broadcasted_iota(jnp.int32, (k, b), 0) \
            + (cb + cb_off) * _CK
        sum_ref[0:1, :] += jnp.sum(jnp.where(cls < n_classes, e, 0.0),
                                   axis=0, keepdims=True)
        out_ref[...] = sum_ref[0:1, :]


def _epilogue_kernel(tc_ref, sc_ref, xlab_ref, margin_ref, out_ref):
    xl = xlab_ref[...]
    e_lab = jnp.exp2(xl * _A - _BB)
    xlc = jnp.clip(xl, -1.0 + 1e-7, 1.0 - 1e-7)
    sin_theta = jnp.sqrt(jnp.maximum(1.0 - xlc * xlc, 0.0))
    mg = margin_ref[...]
    mprime = (xlc * jnp.cos(mg) - sin_theta * jnp.sin(mg)) * _S
    total = tc_ref[...] + jnp.sum(sc_ref[...], axis=0, keepdims=True) \
        - e_lab + jnp.exp2(mprime * _LOG2E - _BB)
    loss = jnp.log(total) + _SHIFT - mprime
    out_ref[...] = jnp.sum(loss, axis=1, keepdims=True) / loss.shape[1]


def _make_sc_part(n_batch):
    span = _K_SC // _NW
    n_chunks = span // _CR
    bpw = n_batch // _NW
    nbv = n_batch // _VEC
    mesh = plsc.VectorSubcoreMesh(core_axis_name="c", subcore_axis_name="s")

    @functools.partial(
        pl.kernel,
        mesh=mesh,
        out_type=(
            jax.ShapeDtypeStruct((_NW * n_batch,), jnp.float32),
            jax.ShapeDtypeStruct((n_batch,), jnp.float32),
        ),
        scratch_types=[
            pltpu.VMEM((2, _CR, n_batch), jnp.float32),
            pltpu.VMEM((n_batch,), jnp.float32),
            pltpu.VMEM((bpw,), jnp.float32),
            pltpu.VMEM((bpw,), jnp.float32),
            pltpu.VMEM((bpw, 8, 128), jnp.float32),
            pltpu.VMEM((bpw,), jnp.float32),
            pltpu.SemaphoreType.DMA((2,)),
            pltpu.SemaphoreType.DMA,
        ],
        compiler_params=pltpu.CompilerParams(needs_layout_passes=False),
    )
    def _sc(x_hbm, g8_hbm, sub8_hbm, sums_hbm, xlab_hbm, buf, acc, g8v,
            sub8v, ltile, xstage, sems, lsem):
        wid = lax.axis_index("s") * _NC + lax.axis_index("c")
        r0 = wid * span
        b32 = wid * bpw
        b0 = (b32 // 128) * 128
        iota = lax.iota(jnp.int32, _VEC)

        pltpu.sync_copy(g8_hbm.at[pl.ds(b32, bpw)], g8v)
        pltpu.sync_copy(sub8_hbm.at[pl.ds(b32, bpw)], sub8v)

        def _scalar_at(ref, r):
            half = ref[pl.ds((r // _VEC) * _VEC, _VEC)]
            return jnp.sum(jnp.where(iota == lax.rem(r, _VEC), half, 0.0))

        @pl.loop(0, bpw)
        def _fire(r):
            g8 = pl.multiple_of(_scalar_at(g8v, r).astype(jnp.int32), 8)
            pltpu.async_copy(x_hbm.at[pl.ds(g8, 8), pl.ds(b0, 128)],
                             ltile.at[r], lsem)

        @pl.loop(0, nbv)
        def _zero(j):
            acc[pl.ds(j * _VEC, _VEC)] = jnp.zeros((_VEC,), jnp.float32)

        pltpu.async_copy(
            x_hbm.at[pl.ds(r0, _CR), :], buf.at[0], sems.at[0])

        @pl.loop(0, n_chunks)
        def _chunks(t):
            slot = lax.rem(t, 2)

            @pl.when(t + 1 < n_chunks)
            def _prefetch():
                nslot = lax.rem(t + 1, 2)
                row = pl.multiple_of(r0 + (t + 1) * _CR, 8)
                pltpu.async_copy(x_hbm.at[pl.ds(row, _CR), :],
                                 buf.at[nslot], sems.at[nslot])

            pltpu.make_async_copy(
                x_hbm.at[pl.ds(r0, _CR), :],
                buf.at[slot], sems.at[slot]).wait()

            @pl.loop(0, nbv)
            def _bv(j):
                zero4 = (jnp.zeros((_VEC,), jnp.float32),) * 4

                def _acc_body(r, c):
                    a0, a1, a2, a3 = c
                    col = pl.ds(j * _VEC, _VEC)
                    a0 = a0 + jnp.exp(buf[slot, r, col] * _S - _SHIFT)
                    a1 = a1 + jnp.exp(buf[slot, r + 1, col] * _S - _SHIFT)
                    a2 = a2 + jnp.exp(buf[slot, r + 2, col] * _S - _SHIFT)
                    a3 = a3 + jnp.exp(buf[slot, r + 3, col] * _S - _SHIFT)
                    return (a0, a1, a2, a3)

                a0, a1, a2, a3 = plsc.parallel_loop(
                    0, _CR, 4, unroll=2, carry=zero4)(_acc_body)
                col = pl.ds(j * _VEC, _VEC)
                acc[col] = acc[col] + ((a0 + a1) + (a2 + a3))

        pltpu.sync_copy(acc, sums_hbm.at[pl.ds(wid * n_batch, n_batch)])

        @pl.loop(0, bpw)
        def _drain(r):
            pltpu.make_async_copy(x_hbm.at[pl.ds(0, 8), pl.ds(0, 128)],
                                  ltile.at[r], lsem).wait()

        zerox = (jnp.zeros((_VEC,), jnp.float32),
                 jnp.zeros((_VEC,), jnp.float32))

        @pl.loop(0, bpw, init_carry=zerox)
        def xvecs(r, xv):
            x0, x1 = xv
            sub8 = _scalar_at(sub8v, r).astype(jnp.int32)
            colv = ((b32 + r - b0) // _VEC) * _VEC
            v = ltile[r, sub8, pl.ds(colv, _VEC)]
            lane_hit = iota == lax.rem(r, _VEC)
            xl_r = jnp.sum(jnp.where(lane_hit, v, 0.0))
            in0 = r // _VEC == 0
            x0 = jnp.where(in0 & lane_hit, xl_r, x0)
            x1 = jnp.where(jnp.logical_not(in0) & lane_hit, xl_r, x1)
            return (x0, x1)

        xstage[pl.ds(0, _VEC)] = xvecs[0]
        xstage[pl.ds(_VEC, _VEC)] = xvecs[1]
        pltpu.sync_copy(xstage, xlab_hbm.at[pl.ds(b32, bpw)])

    return _sc


@jax.jit
def kernel(input, label):
    b, c = input.shape
    xt = input.T
    cb_off = _K_SC // _CK
    n_cb = pl.cdiv(c - _K_SC, _CK)

    margin = _M + _STD * jax.random.normal(jax.random.key(42), (b,),
                                           dtype=jnp.float32)
    valid = label != -1
    margin = jnp.where(valid, margin, 0.0)
    safe_label = jnp.where(valid, label, 0).astype(jnp.int32)

    g8_f = ((safe_label // 8) * 8).astype(jnp.float32)
    sub8_f = (safe_label % 8).astype(jnp.float32)

    sc_parts, sc_xlab = _make_sc_part(b)(xt, g8_f, sub8_f)

    tc_sums = pl.pallas_call(
        functools.partial(_tc_kernel, n_classes=c, cb_off=cb_off),
        grid=(n_cb,),
        in_specs=[pl.BlockSpec((_CK, b), lambda cb: (cb + cb_off, 0))],
        out_specs=pl.BlockSpec((1, b), lambda cb: (0, 0)),
        out_shape=jax.ShapeDtypeStruct((1, b), jnp.float32),
        scratch_shapes=[pltpu.VMEM((8, b), jnp.float32)],
        compiler_params=pltpu.CompilerParams(
            dimension_semantics=("arbitrary",),
        ),
    )(xt)

    loss_mean = pl.pallas_call(
        _epilogue_kernel,
        in_specs=[
            pl.BlockSpec((1, b), lambda: (0, 0)),
            pl.BlockSpec((_NW, b), lambda: (0, 0)),
            pl.BlockSpec((1, b), lambda: (0, 0)),
            pl.BlockSpec((1, b), lambda: (0, 0)),
        ],
        out_specs=pl.BlockSpec((1, 1), lambda: (0, 0)),
        out_shape=jax.ShapeDtypeStruct((1, 1), jnp.float32),
    )(tc_sums, sc_parts.reshape(_NW, b), sc_xlab[None, :], margin[None, :])

    return loss_mean[0, 0]


# speedup vs baseline: 1.0372x; 1.0139x over previous
import functools

import jax
import jax.numpy as jnp
from jax import lax
from jax.experimental import pallas as pl
from jax.experimental.pallas import tpu as pltpu
from jax.experimental.pallas import tpu_sc as plsc

_S = 30.0
_M = 0.5
_STD = 0.0125
_SHIFT = 30.0
_LOG2E = 1.4426950408889634
_A = _S * _LOG2E
_BB = _SHIFT * _LOG2E

_K_SC = 36864
_NC, _NS = 2, 16
_NW = _NC * _NS
_CR = 32
_VEC = 16
_CK = 2048


def _tc_kernel(x_ref, out_ref, sum_ref, *, n_classes, cb_off):
    cb = pl.program_id(0)
    ncb = pl.num_programs(0)

    @pl.when(cb == 0)
    def _init():
        sum_ref[...] = jnp.zeros_like(sum_ref)

    x = x_ref[...]
    k, b = x.shape
    e = jnp.exp2(x * _A - _BB)

    @pl.when(cb != ncb - 1)
    def _body():
        sum_ref[0:1, :] += jnp.sum(e, axis=0, keepdims=True)

    @pl.when(cb == ncb - 1)
    def _last():
        cls = jax.lax.broadcasted_iota(jnp.int32, (k, b), 0) \
            + (cb + cb_off) * _CK
        sum_ref[0:1, :] += jnp.sum(jnp.where(cls < n_classes, e, 0.0),
                                   axis=0, keepdims=True)
        out_ref[...] = sum_ref[0:1, :]


def _epilogue_kernel(tc_ref, sc_ref, xlab_ref, margin_ref, out_ref):
    xl = xlab_ref[...]
    b = xl.shape[1]
    e_lab = jnp.exp2(xl * _A - _BB)
    xlc = jnp.clip(xl, -1.0 + 1e-7, 1.0 - 1e-7)
    sin_theta = jnp.sqrt(jnp.maximum(1.0 - xlc * xlc, 0.0))
    mg = margin_ref[...]
    mprime = (xlc * jnp.cos(mg) - sin_theta * jnp.sin(mg)) * _S
    sc_total = sc_ref[0:1, pl.ds(0, b)]
    for w in range(1, _NW):
        sc_total = sc_total + sc_ref[0:1, pl.ds(w * b, b)]
    total = tc_ref[...] + sc_total - e_lab + jnp.exp2(mprime * _LOG2E - _BB)
    loss = jnp.log(total) + _SHIFT - mprime
    out_ref[...] = jnp.sum(loss, axis=1, keepdims=True) / b


def _make_sc_part(n_batch):
    span = _K_SC // _NW
    n_chunks = span // _CR
    bpw = n_batch // _NW
    nbv = n_batch // _VEC
    mesh = plsc.VectorSubcoreMesh(core_axis_name="c", subcore_axis_name="s")

    @functools.partial(
        pl.kernel,
        mesh=mesh,
        out_type=(
            jax.ShapeDtypeStruct((_NW * n_batch,), jnp.float32),
            jax.ShapeDtypeStruct((n_batch,), jnp.float32),
        ),
        scratch_types=[
            pltpu.VMEM((2, _CR, n_batch), jnp.float32),
            pltpu.VMEM((n_batch,), jnp.float32),
            pltpu.VMEM((bpw,), jnp.float32),
            pltpu.VMEM((bpw,), jnp.float32),
            pltpu.VMEM((bpw, 8, 128), jnp.float32),
            pltpu.VMEM((bpw,), jnp.float32),
            pltpu.SemaphoreType.DMA((2,)),
            pltpu.SemaphoreType.DMA,
        ],
        compiler_params=pltpu.CompilerParams(needs_layout_passes=False),
    )
    def _sc(x_hbm, g8_hbm, sub8_hbm, sums_hbm, xlab_hbm, buf, acc, g8v,
            sub8v, ltile, xstage, sems, lsem):
        wid = lax.axis_index("s") * _NC + lax.axis_index("c")
        r0 = wid * span
        b32 = wid * bpw
        b0 = (b32 // 128) * 128
        iota = lax.iota(jnp.int32, _VEC)

        pltpu.sync_copy(g8_hbm.at[pl.ds(b32, bpw)], g8v)
        pltpu.sync_copy(sub8_hbm.at[pl.ds(b32, bpw)], sub8v)

        def _scalar_at(ref, r):
            half = ref[pl.ds((r // _VEC) * _VEC, _VEC)]
            return jnp.sum(jnp.where(iota == lax.rem(r, _VEC), half, 0.0))

        @pl.loop(0, bpw)
        def _fire(r):
            g8 = pl.multiple_of(_scalar_at(g8v, r).astype(jnp.int32), 8)
            pltpu.async_copy(x_hbm.at[pl.ds(g8, 8), pl.ds(b0, 128)],
                             ltile.at[r], lsem)

        @pl.loop(0, nbv)
        def _zero(j):
            acc[pl.ds(j * _VEC, _VEC)] = jnp.zeros((_VEC,), jnp.float32)

        pltpu.async_copy(
            x_hbm.at[pl.ds(r0, _CR), :], buf.at[0], sems.at[0])

        @pl.loop(0, n_chunks)
        def _chunks(t):
            slot = lax.rem(t, 2)

            @pl.when(t + 1 < n_chunks)
            def _prefetch():
                nslot = lax.rem(t + 1, 2)
                row = pl.multiple_of(r0 + (t + 1) * _CR, 8)
                pltpu.async_copy(x_hbm.at[pl.ds(row, _CR), :],
                                 buf.at[nslot], sems.at[nslot])

            pltpu.make_async_copy(
                x_hbm.at[pl.ds(r0, _CR), :],
                buf.at[slot], sems.at[slot]).wait()

            @pl.loop(0, nbv)
            def _bv(j):
                zero4 = (jnp.zeros((_VEC,), jnp.float32),) * 4

                def _acc_body(r, c):
                    a0, a1, a2, a3 = c
                    col = pl.ds(j * _VEC, _VEC)
                    a0 = a0 + jnp.exp(buf[slot, r, col] * _S - _SHIFT)
                    a1 = a1 + jnp.exp(buf[slot, r + 1, col] * _S - _SHIFT)
                    a2 = a2 + jnp.exp(buf[slot, r + 2, col] * _S - _SHIFT)
                    a3 = a3 + jnp.exp(buf[slot, r + 3, col] * _S - _SHIFT)
                    return (a0, a1, a2, a3)

                a0, a1, a2, a3 = plsc.parallel_loop(
                    0, _CR, 4, unroll=2, carry=zero4)(_acc_body)
                col = pl.ds(j * _VEC, _VEC)
                acc[col] = acc[col] + ((a0 + a1) + (a2 + a3))

        pltpu.sync_copy(acc, sums_hbm.at[pl.ds(wid * n_batch, n_batch)])

        @pl.loop(0, bpw)
        def _drain(r):
            pltpu.make_async_copy(x_hbm.at[pl.ds(0, 8), pl.ds(0, 128)],
                                  ltile.at[r], lsem).wait()

        zerox = (jnp.zeros((_VEC,), jnp.float32),
                 jnp.zeros((_VEC,), jnp.float32))

        @pl.loop(0, bpw, init_carry=zerox)
        def xvecs(r, xv):
            x0, x1 = xv
            sub8 = _scalar_at(sub8v, r).astype(jnp.int32)
            colv = ((b32 + r - b0) // _VEC) * _VEC
            v = ltile[r, sub8, pl.ds(colv, _VEC)]
            lane_hit = iota == lax.rem(r, _VEC)
            xl_r = jnp.sum(jnp.where(lane_hit, v, 0.0))
            in0 = r // _VEC == 0
            x0 = jnp.where(in0 & lane_hit, xl_r, x0)
            x1 = jnp.where(jnp.logical_not(in0) & lane_hit, xl_r, x1)
            return (x0, x1)

        xstage[pl.ds(0, _VEC)] = xvecs[0]
        xstage[pl.ds(_VEC, _VEC)] = xvecs[1]
        pltpu.sync_copy(xstage, xlab_hbm.at[pl.ds(b32, bpw)])

    return _sc


@jax.jit
def kernel(input, label):
    b, c = input.shape
    xt = input.T
    cb_off = _K_SC // _CK
    n_cb = pl.cdiv(c - _K_SC, _CK)

    margin = _M + _STD * jax.random.normal(jax.random.key(42), (b,),
                                           dtype=jnp.float32)
    valid = label != -1
    margin = jnp.where(valid, margin, 0.0)
    safe_label = jnp.where(valid, label, 0).astype(jnp.int32)

    g8_f = ((safe_label // 8) * 8).astype(jnp.float32)
    sub8_f = (safe_label % 8).astype(jnp.float32)

    sc_parts, sc_xlab = _make_sc_part(b)(xt, g8_f, sub8_f)

    tc_sums = pl.pallas_call(
        functools.partial(_tc_kernel, n_classes=c, cb_off=cb_off),
        grid=(n_cb,),
        in_specs=[pl.BlockSpec((_CK, b), lambda cb: (cb + cb_off, 0))],
        out_specs=pl.BlockSpec((1, b), lambda cb: (0, 0)),
        out_shape=jax.ShapeDtypeStruct((1, b), jnp.float32),
        scratch_shapes=[pltpu.VMEM((8, b), jnp.float32)],
        compiler_params=pltpu.CompilerParams(
            dimension_semantics=("arbitrary",),
        ),
    )(xt)

    loss_mean = pl.pallas_call(
        _epilogue_kernel,
        in_specs=[
            pl.BlockSpec((1, b), lambda: (0, 0)),
            pl.BlockSpec((1, _NW * b), lambda: (0, 0)),
            pl.BlockSpec((1, b), lambda: (0, 0)),
            pl.BlockSpec((1, b), lambda: (0, 0)),
        ],
        out_specs=pl.BlockSpec((1, 1), lambda: (0, 0)),
        out_shape=jax.ShapeDtypeStruct((1, 1), jnp.float32),
    )(tc_sums, sc_parts[None, :], sc_xlab[None, :], margin[None, :])

    return loss_mean[0, 0]
